# baseline (speedup 1.0000x reference)
import jax
import jax.numpy as jnp
import numpy as np
from jax import lax
from jax.experimental import pallas as pl
from jax.experimental.pallas import tpu as pltpu
from jax.experimental.pallas import tpu_sc as plsc

B = 16384
F = 26
V = 100000
D = 32
NUM = 13
EPS = 1e-5

NW = 32
BPW = B // NW
NT = BPW // 8
SLOTS = 224
RPW = NT * SLOTS
R = NW * RPW
IR = RPW // 128
CH = 1024
NCH = RPW // CH

GF = 7
VC4 = 16384
VSH = VC4.bit_length() - 1
DST = -(-V // VC4)
TROWS = GF * DST * VC4 * 4


def _slot_tables():
    t = np.arange(RPW)
    bt, r = t // SLOTS, t % SLOTS
    jt, bb, ff = r // 32, (r // 4) % 8, r % 4
    j = 4 * jt + ff
    b = bt * 8 + bb
    valid = j < F
    pos = np.where(valid, b * F + np.minimum(j, F - 1), 0)
    voff = np.where(valid, (j >> 2) * (DST * VC4 * 4) + (j & 3), 0)
    pos_all = pos[None, :] + (np.arange(NW) * (BPW * F))[:, None]
    return (pos_all.reshape(NW, IR, 128).astype(np.int32),
            voff.reshape(IR, 128).astype(np.int32))


_POS, _VOFF = _slot_tables()


def _gather_body(tab_hbm, xcat_hbm, pos_hbm, voff_hbm,
                 out_hbm, pos_v, idx_v, voff_v, rows_v, gsem):
    wid = lax.axis_index("s") * 2 + lax.axis_index("c")

    pltpu.sync_copy(pos_hbm.at[wid], pos_v)
    pltpu.sync_copy(voff_hbm, voff_v)

    def pchunk(c, carry):
        copies = []
        for jj in range(8):
            r = c * 8 + jj
            cp = pltpu.make_async_copy(
                xcat_hbm.at[pos_v.at[r]], idx_v.at[r], gsem)
            cp.start()
            copies.append(cp)
        for cp in copies:
            cp.wait()
        return carry

    lax.fori_loop(0, NCH, pchunk, 0)

    def fix_row(r, carry):
        for cc in range(8):
            sl = pl.ds(cc * 16, 16)
            rv = idx_v[r, sl]
            scr = lax.shift_left(lax.shift_right_logical(rv, VSH), VSH + 2) + \
                lax.shift_left(rv & (VC4 - 1), 2)
            idx_v[r, sl] = scr + voff_v[r, sl]
        return carry

    lax.fori_loop(0, IR, fix_row, 0)

    base = wid * RPW

    def chunk(c, carry):
        copies = []
        for jj in range(8):
            cp = pltpu.make_async_copy(
                tab_hbm.at[idx_v.at[c * 8 + jj]],
                rows_v.at[pl.ds(jj * 128, 128)],
                gsem,
            )
            cp.start()
            copies.append(cp)
        for cp in copies:
            cp.wait()
        pltpu.sync_copy(rows_v, out_hbm.at[pl.ds(base + c * CH, CH)])
        return carry

    lax.fori_loop(0, NCH, chunk, 0)


def _sc_gather(tab2, xcat_flat, pos, voff):
    mesh = plsc.VectorSubcoreMesh(core_axis_name="c", subcore_axis_name="s")
    k = pl.kernel(
        _gather_body,
        out_type=jax.ShapeDtypeStruct((R, D), jnp.float32),
        mesh=mesh,
        scratch_types=[
            pltpu.VMEM((IR, 128), jnp.int32),
            pltpu.VMEM((IR, 128), jnp.int32),
            pltpu.VMEM((IR, 128), jnp.int32),
            pltpu.VMEM((CH, D), jnp.float32),
            pltpu.SemaphoreType.DMA,
        ],
        compiler_params=pltpu.CompilerParams(use_tc_tiling_on_sc=False),
    )
    return k(tab2, xcat_flat, pos, voff)


def _detile_body(in_ref, out_ref):
    x = in_ref[...]
    out_ref[...] = x.reshape(128, VC4).T


def _tc_detile(tab_t):
    return pl.pallas_call(
        _detile_body,
        grid=(GF, DST),
        in_specs=[pl.BlockSpec((4, D, VC4), lambda g, s: (g, 0, s))],
        out_specs=pl.BlockSpec((VC4, 128), lambda g, s: (g * DST + s, 0)),
        out_shape=jax.ShapeDtypeStruct((TROWS * D // 128, 128), jnp.float32),
    )(tab_t)


BT = 1024


def _mlp_body(x_ref, num_ref, w1_ref, w1n_ref, w2_ref, par_ref, out_ref):
    p = par_ref[...]
    b1 = p[0:1, :]
    a1 = p[1:2, :] * lax.rsqrt(p[4:5, :] + EPS)
    c1 = p[2:3, :] - p[3:4, :] * a1
    b2 = p[5:6, :]
    a2 = p[6:7, :] * lax.rsqrt(p[9:10, :] + EPS)
    c2 = p[7:8, :] - p[8:9, :] * a2
    w3 = p[10:11, :]
    b3 = p[11:12, 0:1]

    x = x_ref[...]
    z = jnp.dot(num_ref[...], w1n_ref[...], preferred_element_type=jnp.float32)
    for jt in range(7):
        xj = x[:, jt].reshape(BT, 128)
        z = z + jnp.dot(xj, w1_ref[jt], preferred_element_type=jnp.float32)
    h = jnp.maximum(z + b1, 0.0) * a1 + c1
    z2 = jnp.dot(h, w2_ref[...], preferred_element_type=jnp.float32)
    h2 = jnp.maximum(z2 + b2, 0.0) * a2 + c2
    z3 = jnp.sum(h2 * w3, axis=1, keepdims=True) + b3
    out_ref[...] = jax.nn.sigmoid(z3)


def _tc_mlp(x4, xnum_p, w1_st, w1n_t, w2_t, params):
    return pl.pallas_call(
        _mlp_body,
        grid=(B // BT,),
        in_specs=[
            pl.BlockSpec((BT // 8, 7, 8, 128), lambda i: (i, 0, 0, 0)),
            pl.BlockSpec((BT, 128), lambda i: (i, 0)),
            pl.BlockSpec((7, 128, 64), lambda i: (0, 0, 0)),
            pl.BlockSpec((128, 64), lambda i: (0, 0)),
            pl.BlockSpec((64, 64), lambda i: (0, 0)),
            pl.BlockSpec((16, 64), lambda i: (0, 0)),
        ],
        out_specs=pl.BlockSpec((BT, 1), lambda i: (i, 0)),
        out_shape=jax.ShapeDtypeStruct((B, 1), jnp.float32),
    )(x4, xnum_p, w1_st, w1n_t, w2_t, params)


def kernel(X_cat, X_num, tables, W1, b1, g1, be1, rm1, rv1,
           W2, b2, g2, be2, rm2, rv2, W3, b3):
    tab_t = jnp.transpose(tables, (0, 2, 1))
    tab2 = _tc_detile(tab_t).reshape(TROWS, D)
    xcat_flat = X_cat.reshape(B * F)
    pos = jnp.asarray(_POS)
    voff = jnp.asarray(_VOFF)

    gathered = _sc_gather(tab2, xcat_flat, pos, voff)
    x4 = gathered.reshape(B // 8, 7, 8, 128)

    xnum_p = jnp.pad(X_num, ((0, 0), (0, 128 - NUM)))
    w1_st = jnp.pad(W1[:, : F * D].T, ((0, 64), (0, 0))).reshape(7, 128, 64)
    w1n_t = jnp.pad(W1[:, F * D:], ((0, 0), (0, 128 - NUM))).T
    w2_t = W2.T
    params = jnp.zeros((16, 64), jnp.float32)
    rows = [b1, g1, be1, rm1, rv1, b2, g2, be2, rm2, rv2,
            W3[0], jnp.full((64,), b3[0], jnp.float32)]
    params = params.at[: len(rows)].set(jnp.stack(rows))

    return _tc_mlp(x4, xnum_p, w1_st, w1n_t, w2_t, params)


# speedup vs baseline: 1.0251x; 1.0251x over previous
import jax
import jax.numpy as jnp
import numpy as np
from jax import lax
from jax.experimental import pallas as pl
from jax.experimental.pallas import tpu as pltpu
from jax.experimental.pallas import tpu_sc as plsc

B = 16384
F = 26
V = 100000
D = 32
NUM = 13
EPS = 1e-5

NW = 32
BPW = B // NW
NT = BPW // 8
SLOTS = 224
RPW = NT * SLOTS
R = NW * RPW
IR = RPW // 128
CH = 1024
NCH = RPW // CH

GF = 7
VC4 = 8192
VSH = VC4.bit_length() - 1
DST = -(-V // VC4)
TROWS = GF * DST * VC4 * 4


def _slot_tables():
    t = np.arange(RPW)
    bt, r = t // SLOTS, t % SLOTS
    jt, bb, ff = r // 32, (r // 4) % 8, r % 4
    j = 4 * jt + ff
    b = bt * 8 + bb
    valid = j < F
    pos = np.where(valid, b * F + np.minimum(j, F - 1), 0)
    voff = np.where(valid, (j >> 2) * (DST * VC4 * 4) + (j & 3), 0)
    pos_all = pos[None, :] + (np.arange(NW) * (BPW * F))[:, None]
    return (pos_all.reshape(NW, IR, 128).astype(np.int32),
            voff.reshape(IR, 128).astype(np.int32))


_POS, _VOFF = _slot_tables()


def _gather_body(tab_hbm, xcat_hbm, pos_hbm, voff_hbm,
                 out_hbm, pos_v, idx_v, voff_v, rows_v, rows2_v, gsem, gsem2):
    wid = lax.axis_index("s") * 2 + lax.axis_index("c")

    pltpu.sync_copy(pos_hbm.at[wid], pos_v)
    pltpu.sync_copy(voff_hbm, voff_v)

    def pchunk(c, carry):
        copies = []
        for jj in range(8):
            r = c * 8 + jj
            cp = pltpu.make_async_copy(
                xcat_hbm.at[pos_v.at[r]], idx_v.at[r], gsem)
            cp.start()
            copies.append(cp)
        for cp in copies:
            cp.wait()
        return carry

    lax.fori_loop(0, NCH, pchunk, 0)

    def fix_row(r, carry):
        for cc in range(8):
            sl = pl.ds(cc * 16, 16)
            rv = idx_v[r, sl]
            scr = lax.shift_left(lax.shift_right_logical(rv, VSH), VSH + 2) + \
                lax.shift_left(rv & (VC4 - 1), 2)
            idx_v[r, sl] = scr + voff_v[r, sl]
        return carry

    lax.fori_loop(0, IR, fix_row, 0)

    base = wid * RPW

    def g8(c, buf, sem, go):
        for jj in range(8):
            cp = pltpu.make_async_copy(
                tab_hbm.at[idx_v.at[c * 8 + jj]],
                buf.at[pl.ds(jj * 128, 128)],
                sem,
            )
            if go:
                cp.start()
            else:
                cp.wait()

    g8(0, rows_v, gsem, True)

    def chunk2(c2, carry):
        c0 = 2 * c2
        g8(c0, rows_v, gsem, False)
        g8(c0 + 1, rows2_v, gsem2, True)
        pltpu.sync_copy(rows_v, out_hbm.at[pl.ds(base + c0 * CH, CH)])
        g8(c0 + 1, rows2_v, gsem2, False)

        @pl.when(c2 < NCH // 2 - 1)
        def _():
            g8(c0 + 2, rows_v, gsem, True)

        pltpu.sync_copy(rows2_v, out_hbm.at[pl.ds(base + (c0 + 1) * CH, CH)])
        return carry

    lax.fori_loop(0, NCH // 2, chunk2, 0)


def _sc_gather(tab2, xcat_flat, pos, voff):
    mesh = plsc.VectorSubcoreMesh(core_axis_name="c", subcore_axis_name="s")
    k = pl.kernel(
        _gather_body,
        out_type=jax.ShapeDtypeStruct((R, D), jnp.float32),
        mesh=mesh,
        scratch_types=[
            pltpu.VMEM((IR, 128), jnp.int32),
            pltpu.VMEM((IR, 128), jnp.int32),
            pltpu.VMEM((IR, 128), jnp.int32),
            pltpu.VMEM((CH, D), jnp.float32),
            pltpu.VMEM((CH, D), jnp.float32),
            pltpu.SemaphoreType.DMA,
            pltpu.SemaphoreType.DMA,
        ],
        compiler_params=pltpu.CompilerParams(use_tc_tiling_on_sc=False),
    )
    return k(tab2, xcat_flat, pos, voff)


def _detile_body(in_ref, out_ref):
    x = in_ref[...]
    out_ref[...] = x.reshape(128, VC4).T


def _tc_detile(tab_t):
    return pl.pallas_call(
        _detile_body,
        grid=(GF, DST),
        in_specs=[pl.BlockSpec((4, D, VC4), lambda g, s: (g, 0, s))],
        out_specs=pl.BlockSpec((VC4, 128), lambda g, s: (g * DST + s, 0)),
        out_shape=jax.ShapeDtypeStruct((TROWS * D // 128, 128), jnp.float32),
    )(tab_t)


BT = 1024


def _mlp_body(x_ref, num_ref, w1_ref, w1n_ref, w2_ref, par_ref, out_ref):
    p = par_ref[...]
    b1 = p[0:1, :]
    a1 = p[1:2, :] * lax.rsqrt(p[4:5, :] + EPS)
    c1 = p[2:3, :] - p[3:4, :] * a1
    b2 = p[5:6, :]
    a2 = p[6:7, :] * lax.rsqrt(p[9:10, :] + EPS)
    c2 = p[7:8, :] - p[8:9, :] * a2
    w3 = p[10:11, :]
    b3 = p[11:12, 0:1]

    x = x_ref[...]
    z = jnp.dot(num_ref[...], w1n_ref[...], preferred_element_type=jnp.float32)
    for jt in range(7):
        xj = x[:, jt].reshape(BT, 128)
        z = z + jnp.dot(xj, w1_ref[jt], preferred_element_type=jnp.float32)
    h = jnp.maximum(z + b1, 0.0) * a1 + c1
    z2 = jnp.dot(h, w2_ref[...], preferred_element_type=jnp.float32)
    h2 = jnp.maximum(z2 + b2, 0.0) * a2 + c2
    z3 = jnp.sum(h2 * w3, axis=1, keepdims=True) + b3
    out_ref[...] = jax.nn.sigmoid(z3)


def _tc_mlp(x4, xnum_p, w1_st, w1n_t, w2_t, params):
    return pl.pallas_call(
        _mlp_body,
        grid=(B // BT,),
        in_specs=[
            pl.BlockSpec((BT // 8, 7, 8, 128), lambda i: (i, 0, 0, 0)),
            pl.BlockSpec((BT, 128), lambda i: (i, 0)),
            pl.BlockSpec((7, 128, 64), lambda i: (0, 0, 0)),
            pl.BlockSpec((128, 64), lambda i: (0, 0)),
            pl.BlockSpec((64, 64), lambda i: (0, 0)),
            pl.BlockSpec((16, 64), lambda i: (0, 0)),
        ],
        out_specs=pl.BlockSpec((BT, 1), lambda i: (i, 0)),
        out_shape=jax.ShapeDtypeStruct((B, 1), jnp.float32),
    )(x4, xnum_p, w1_st, w1n_t, w2_t, params)


def kernel(X_cat, X_num, tables, W1, b1, g1, be1, rm1, rv1,
           W2, b2, g2, be2, rm2, rv2, W3, b3):
    tab_t = jnp.transpose(tables, (0, 2, 1))
    tab2 = _tc_detile(tab_t).reshape(TROWS, D)
    xcat_flat = X_cat.reshape(B * F)
    pos = jnp.asarray(_POS)
    voff = jnp.asarray(_VOFF)

    gathered = _sc_gather(tab2, xcat_flat, pos, voff)
    x4 = gathered.reshape(B // 8, 7, 8, 128)

    xnum_p = jnp.pad(X_num, ((0, 0), (0, 128 - NUM)))
    w1_st = jnp.pad(W1[:, : F * D].T, ((0, 64), (0, 0))).reshape(7, 128, 64)
    w1n_t = jnp.pad(W1[:, F * D:], ((0, 0), (0, 128 - NUM))).T
    w2_t = W2.T
    params = jnp.zeros((16, 64), jnp.float32)
    rows = [b1, g1, be1, rm1, rv1, b2, g2, be2, rm2, rv2,
            W3[0], jnp.full((64,), b3[0], jnp.float32)]
    params = params.at[: len(rows)].set(jnp.stack(rows))

    return _tc_mlp(x4, xnum_p, w1_st, w1n_t, w2_t, params)
